# baseline (speedup 1.0000x reference)
import functools

import jax
import jax.numpy as jnp
from jax import lax
from jax.experimental import pallas as pl
from jax.experimental.pallas import tpu as pltpu
from jax.experimental.pallas import tpu_sc as plsc

D = 64
N = 819200
W = 128
NW = 32
PW = N // NW
TS = PW // W
SCALE = 8.0


def _sc_gather(lut, xin):
    mesh = plsc.VectorSubcoreMesh(core_axis_name="c", subcore_axis_name="s")

    @functools.partial(
        pl.kernel,
        out_type=jax.ShapeDtypeStruct((N, D), jnp.float32),
        mesh=mesh,
        scratch_types=[
            pltpu.VMEM((PW,), jnp.int32),
            pltpu.VMEM((2, W, D), jnp.float32),
            pltpu.VMEM((2, W, D), jnp.float32),
            pltpu.SemaphoreType.DMA,
            pltpu.SemaphoreType.DMA,
            pltpu.SemaphoreType.DMA,
            pltpu.SemaphoreType.DMA,
            pltpu.SemaphoreType.DMA,
        ],
        compiler_params=pltpu.CompilerParams(
            use_tc_tiling_on_sc=False, skip_device_barrier=True),
    )
    def k(lut_hbm, x_hbm, out_hbm, idx_all, g, tr, gs0, gs1, ws0, ws1, isem):
        w = lax.axis_index("s") * 2 + lax.axis_index("c")
        base = w * PW
        gsem = (gs0, gs1)
        wsem = (ws0, ws1)

        pltpu.async_copy(x_hbm.at[0, pl.ds(base, PW)], idx_all, isem).wait()

        def start_gather(step, q):
            pltpu.make_async_copy(
                lut_hbm.at[idx_all.at[pl.ds(step * W, W)]], g.at[q],
                gsem[q]).start()

        def wait_gather(step, q):
            pltpu.make_async_copy(
                lut_hbm.at[idx_all.at[pl.ds(step * W, W)]], g.at[q],
                gsem[q]).wait()

        def start_write(step, q):
            pltpu.make_async_copy(
                tr.at[q], out_hbm.at[pl.ds(base + step * W, W), :],
                wsem[q]).start()

        def wait_write(step, q):
            pltpu.make_async_copy(
                tr.at[q], out_hbm.at[pl.ds(base + step * W, W), :],
                wsem[q]).wait()

        def scale(q):
            @plsc.parallel_loop(0, W, step=1, unroll=8)
            def _(j):
                for c in range(0, D, 16):
                    v = g[q, j, pl.ds(c, 16)]
                    tr.at[q, j, pl.ds(c, 16)][...] = v * SCALE

        start_gather(0, 0)

        @pl.loop(0, TS, step=2)
        def _(k0):
            for p in (0, 1):
                kk = k0 + p
                q = 1 - p

                @pl.when(kk < TS - 1)
                def _():
                    start_gather(kk + 1, q)

                wait_gather(kk, p)

                @pl.when(kk >= 2)
                def _():
                    wait_write(kk - 2, p)

                scale(p)
                start_write(kk, p)

        wait_write(TS - 2, 0)
        wait_write(TS - 1, 1)

    return k(lut, xin)


def kernel(x, lut):
    xin = x.reshape(1, N)
    out = _sc_gather(lut, xin)
    return out.reshape(x.shape[0], x.shape[1], D)


# speedup vs baseline: 1.0008x; 1.0008x over previous
import functools

import jax
import jax.numpy as jnp
from jax import lax
from jax.experimental import pallas as pl
from jax.experimental.pallas import tpu as pltpu
from jax.experimental.pallas import tpu_sc as plsc

D = 64
N = 819200
W = 128
NW = 32
PW = N // NW
TS = PW // W
SCALE = 8.0


def _sc_gather(lut, xin):
    mesh = plsc.VectorSubcoreMesh(core_axis_name="c", subcore_axis_name="s")

    @functools.partial(
        pl.kernel,
        out_type=jax.ShapeDtypeStruct((N, D), jnp.float32),
        mesh=mesh,
        scratch_types=[
            pltpu.VMEM((PW,), jnp.int32),
            pltpu.VMEM((2, W, D), jnp.float32),
            pltpu.VMEM((2, W, D), jnp.float32),
            pltpu.SemaphoreType.DMA,
            pltpu.SemaphoreType.DMA,
            pltpu.SemaphoreType.DMA,
            pltpu.SemaphoreType.DMA,
            pltpu.SemaphoreType.DMA,
        ],
        compiler_params=pltpu.CompilerParams(use_tc_tiling_on_sc=False),
    )
    def k(lut_hbm, x_hbm, out_hbm, idx_all, g, tr, gs0, gs1, ws0, ws1, isem):
        w = lax.axis_index("s") * 2 + lax.axis_index("c")
        base = w * PW
        gsem = (gs0, gs1)
        wsem = (ws0, ws1)

        pltpu.async_copy(x_hbm.at[0, pl.ds(base, PW)], idx_all, isem).wait()

        def start_gather(step, q):
            pltpu.make_async_copy(
                lut_hbm.at[idx_all.at[pl.ds(step * W, W)]], g.at[q],
                gsem[q]).start()

        def wait_gather(step, q):
            pltpu.make_async_copy(
                lut_hbm.at[idx_all.at[pl.ds(step * W, W)]], g.at[q],
                gsem[q]).wait()

        def start_write(step, q):
            pltpu.make_async_copy(
                tr.at[q], out_hbm.at[pl.ds(base + step * W, W), :],
                wsem[q]).start()

        def wait_write(step, q):
            pltpu.make_async_copy(
                tr.at[q], out_hbm.at[pl.ds(base + step * W, W), :],
                wsem[q]).wait()

        def scale(q):
            @plsc.parallel_loop(0, W, step=1, unroll=8)
            def _(j):
                for c in range(0, D, 16):
                    v = g[q, j, pl.ds(c, 16)]
                    tr.at[q, j, pl.ds(c, 16)][...] = v * SCALE

        start_gather(0, 0)

        @pl.loop(0, TS, step=2)
        def _(k0):
            for p in (0, 1):
                kk = k0 + p
                q = 1 - p

                @pl.when(kk < TS - 1)
                def _():
                    start_gather(kk + 1, q)

                wait_gather(kk, p)

                @pl.when(kk >= 2)
                def _():
                    wait_write(kk - 2, p)

                scale(p)
                start_write(kk, p)

        wait_write(TS - 2, 0)
        wait_write(TS - 1, 1)

    return k(lut, xin)


def kernel(x, lut):
    xin = x.reshape(1, N)
    out = _sc_gather(lut, xin)
    return out.reshape(x.shape[0], x.shape[1], D)
